# Initial kernel scaffold; baseline (speedup 1.0000x reference)
import jax
import jax.numpy as jnp
from jax.experimental import pallas as pl


def kernel(node_emb, edge_index, W1, b1, W2, b2, Wg, bg):
    raise NotImplementedError("write your pallas kernel here")


# speedup vs baseline: 3.6501x; 3.6501x over previous
import functools

import jax
import jax.numpy as jnp
from jax import lax
from jax.experimental import pallas as pl
from jax.experimental.pallas import tpu as pltpu
from jax.experimental.pallas import tpu_sc as plsc

N_NODES = 10000
E = 320000
D = 128
LANES = 16

NC, NS = 2, 16
NW = NC * NS
EPW = E // NW
CHUNK = 80
NCHUNK = EPW // CHUNK
N_PAD = 10240
RPT = N_PAD // NS
SUBR = 64
NSUB = RPT // SUBR
CW = 128

_MESH = plsc.VectorSubcoreMesh(
    core_axis_name="c", subcore_axis_name="s", num_cores=NC, num_subcores=NS
)


def _ab_body(x_ref, w1a_ref, w1b_ref, a_ref, b_ref):
    x = x_ref[...]
    a_ref[...] = jnp.dot(x, w1a_ref[...], preferred_element_type=jnp.float32)
    b_ref[...] = jnp.dot(x, w1b_ref[...], preferred_element_type=jnp.float32)


def _node_proj(node_emb, w1a, w1b):
    blk = 1000
    grid = N_NODES // blk
    return pl.pallas_call(
        _ab_body,
        grid=(grid,),
        in_specs=[
            pl.BlockSpec((blk, D), lambda i: (i, 0)),
            pl.BlockSpec((D, D), lambda i: (0, 0)),
            pl.BlockSpec((D, D), lambda i: (0, 0)),
        ],
        out_specs=[
            pl.BlockSpec((blk, D), lambda i: (i, 0)),
            pl.BlockSpec((blk, D), lambda i: (i, 0)),
        ],
        out_shape=[
            jax.ShapeDtypeStruct((N_NODES, D), jnp.float32),
            jax.ShapeDtypeStruct((N_NODES, D), jnp.float32),
        ],
    )(node_emb, w1a, w1b)


@functools.partial(
    pl.kernel,
    out_type=jax.ShapeDtypeStruct((E, D), jnp.float32),
    mesh=_MESH,
    scratch_types=[
        pltpu.VMEM((CHUNK,), jnp.int32),
        pltpu.VMEM((CHUNK,), jnp.int32),
        pltpu.VMEM((CHUNK, D), jnp.float32),
        pltpu.VMEM((CHUNK, D), jnp.float32),
        pltpu.SemaphoreType.DMA,
        pltpu.SemaphoreType.DMA,
    ],
)
def _gather_kernel(a_hbm, b_hbm, src_hbm, dst_hbm, out_hbm,
                   si, di, ba, bb, sema, semb):
    wid = lax.axis_index("s") * NC + lax.axis_index("c")
    base = wid * EPW

    @pl.loop(0, NCHUNK)
    def _chunk(g):
        off = base + g * CHUNK
        pltpu.sync_copy(src_hbm.at[pl.ds(off, CHUNK)], si)
        pltpu.sync_copy(dst_hbm.at[pl.ds(off, CHUNK)], di)
        cp_a = pltpu.async_copy(a_hbm.at[si], ba, sema)
        cp_b = pltpu.async_copy(b_hbm.at[di], bb, semb)
        cp_a.wait()
        cp_b.wait()

        @pl.loop(0, CHUNK)
        def _row(r):
            for j in range(D // LANES):
                sl = pl.ds(j * LANES, LANES)
                ba[r, sl] = ba[r, sl] + bb[r, sl]

        pltpu.sync_copy(ba, out_hbm.at[pl.ds(off, CHUNK)])


def _mlp_body(h0_ref, b1_ref, w2_ref, b2_ref, wg_ref, bg_ref, out_ref):
    h = jnp.maximum(h0_ref[...] + b1_ref[...], 0.0)
    inter = jnp.dot(h, w2_ref[...], preferred_element_type=jnp.float32) + b2_ref[...]
    gate = jax.nn.sigmoid(
        jnp.dot(inter, wg_ref[...], preferred_element_type=jnp.float32) + bg_ref[...]
    )
    out_ref[...] = inter * gate


def _edge_mlp(h0, b1, w2, b2, wg, bg):
    blk = 3200
    grid = E // blk
    vec = lambda i: (0, 0)
    return pl.pallas_call(
        _mlp_body,
        grid=(grid,),
        in_specs=[
            pl.BlockSpec((blk, D), lambda i: (i, 0)),
            pl.BlockSpec((1, D), vec),
            pl.BlockSpec((D, D), vec),
            pl.BlockSpec((1, D), vec),
            pl.BlockSpec((D, D), vec),
            pl.BlockSpec((1, D), vec),
        ],
        out_specs=pl.BlockSpec((blk, D), lambda i: (i, 0)),
        out_shape=jax.ShapeDtypeStruct((E, D), jnp.float32),
    )(h0, b1.reshape(1, D), w2, b2.reshape(1, D), wg, bg.reshape(1, D))


@functools.partial(
    pl.kernel,
    out_type=jax.ShapeDtypeStruct((NC, N_PAD, D), jnp.float32),
    mesh=_MESH,
    scratch_types=[
        pltpu.VMEM((CHUNK,), jnp.int32),
        pltpu.VMEM((CHUNK, D), jnp.float32),
        pltpu.VMEM((SUBR, D), jnp.float32),
        pltpu.VMEM_SHARED((N_PAD, D), jnp.float32),
    ],
)
def _scatter_kernel(gated_hbm, src_hbm, psum_hbm, idx_v, rows_v, stage_v, accum):
    cid = lax.axis_index("c")
    sid = lax.axis_index("s")
    wid = sid * NC + cid
    rbase = sid * RPT

    zeros16 = jnp.zeros((LANES,), jnp.float32)

    @pl.loop(0, SUBR)
    def _zero(r):
        for j in range(D // LANES):
            stage_v[r, pl.ds(j * LANES, LANES)] = zeros16

    @pl.loop(0, NSUB)
    def _zinit(k):
        pltpu.sync_copy(stage_v, accum.at[pl.ds(rbase + k * SUBR, SUBR)])

    plsc.subcore_barrier()

    base = wid * EPW

    @pl.loop(0, NCHUNK)
    def _chunk(g):
        off = base + g * CHUNK
        pltpu.sync_copy(src_hbm.at[pl.ds(off, CHUNK)], idx_v)
        pltpu.sync_copy(gated_hbm.at[pl.ds(off, CHUNK)], rows_v)
        pltpu.sync_copy(rows_v, accum.at[idx_v], add=True)

    plsc.subcore_barrier()

    @pl.loop(0, NSUB)
    def _wb(k):
        r0 = rbase + k * SUBR
        pltpu.sync_copy(accum.at[pl.ds(r0, SUBR)], stage_v)
        pltpu.sync_copy(stage_v, psum_hbm.at[cid, pl.ds(r0, SUBR)])


@functools.partial(
    pl.kernel,
    out_type=jax.ShapeDtypeStruct((NC, N_PAD, CW), jnp.float32),
    mesh=_MESH,
    scratch_types=[
        pltpu.VMEM((CHUNK,), jnp.int32),
        pltpu.VMEM((CHUNK, CW), jnp.float32),
        pltpu.VMEM((SUBR, CW), jnp.float32),
        pltpu.VMEM_SHARED((N_PAD, CW), jnp.float32),
    ],
)
def _count_kernel(src_hbm, pcnt_hbm, idx_v, ones_v, stage_c, accum_c):
    cid = lax.axis_index("c")
    sid = lax.axis_index("s")
    wid = sid * NC + cid
    rbase = sid * RPT

    zeros16 = jnp.zeros((LANES,), jnp.float32)
    ones16 = jnp.ones((LANES,), jnp.float32)

    @pl.loop(0, SUBR)
    def _zero(r):
        for j in range(CW // LANES):
            stage_c[r, pl.ds(j * LANES, LANES)] = zeros16

    @pl.loop(0, CHUNK)
    def _fill_ones(r):
        for j in range(CW // LANES):
            ones_v[r, pl.ds(j * LANES, LANES)] = ones16

    @pl.loop(0, NSUB)
    def _zinit(k):
        pltpu.sync_copy(stage_c, accum_c.at[pl.ds(rbase + k * SUBR, SUBR)])

    plsc.subcore_barrier()

    base = wid * EPW

    @pl.loop(0, NCHUNK)
    def _chunk(g):
        off = base + g * CHUNK
        pltpu.sync_copy(src_hbm.at[pl.ds(off, CHUNK)], idx_v)
        pltpu.sync_copy(ones_v, accum_c.at[idx_v], add=True)

    plsc.subcore_barrier()

    @pl.loop(0, NSUB)
    def _wb(k):
        r0 = rbase + k * SUBR
        pltpu.sync_copy(accum_c.at[pl.ds(r0, SUBR)], stage_c)
        pltpu.sync_copy(stage_c, pcnt_hbm.at[cid, pl.ds(r0, SUBR)])


def _fin_body(p0_ref, p1_ref, c0_ref, c1_ref, out_ref):
    cnt = c0_ref[:, 0:1] + c1_ref[:, 0:1]
    out_ref[...] = (p0_ref[...] + p1_ref[...]) / jnp.maximum(cnt, 1.0)


def _finalize(psum, pcnt):
    blk = 1024
    grid = N_PAD // blk
    return pl.pallas_call(
        _fin_body,
        grid=(grid,),
        in_specs=[
            pl.BlockSpec((blk, D), lambda i: (i, 0)),
            pl.BlockSpec((blk, D), lambda i: (i, 0)),
            pl.BlockSpec((blk, CW), lambda i: (i, 0)),
            pl.BlockSpec((blk, CW), lambda i: (i, 0)),
        ],
        out_specs=pl.BlockSpec((blk, D), lambda i: (i, 0)),
        out_shape=jax.ShapeDtypeStruct((N_PAD, D), jnp.float32),
    )(psum[0], psum[1], pcnt[0], pcnt[1])


def kernel(node_emb, edge_index, W1, b1, W2, b2, Wg, bg):
    src = edge_index[0]
    dst = edge_index[1]
    a, b = _node_proj(node_emb, W1[:D], W1[D:])
    h0 = _gather_kernel(a, b, src, dst)
    gated = _edge_mlp(h0, b1, W2, b2, Wg, bg)
    psum = _scatter_kernel(gated, src)
    pcnt = _count_kernel(src)
    return _finalize(psum, pcnt)[:N_NODES]


# speedup vs baseline: 5.0840x; 1.3928x over previous
import functools

import jax
import jax.numpy as jnp
from jax import lax
from jax.experimental import pallas as pl
from jax.experimental.pallas import tpu as pltpu
from jax.experimental.pallas import tpu_sc as plsc

N_NODES = 10000
E = 320000
D = 128
LANES = 16

NC, NS = 2, 16
NW = NC * NS
EPW = E // NW
CHUNK = 80
NCHUNK = EPW // CHUNK
N_PAD = 10240
RPT = N_PAD // NS
SUBR = 64
NSUB = RPT // SUBR
CW = 128

_MESH = plsc.VectorSubcoreMesh(
    core_axis_name="c", subcore_axis_name="s", num_cores=NC, num_subcores=NS
)


def _ab_body(x_ref, w1a_ref, w1b_ref, a_ref, b_ref):
    x = x_ref[...]
    a_ref[...] = jnp.dot(x, w1a_ref[...], preferred_element_type=jnp.float32)
    b_ref[...] = jnp.dot(x, w1b_ref[...], preferred_element_type=jnp.float32)


def _node_proj(node_emb, w1a, w1b):
    blk = 1000
    grid = N_NODES // blk
    return pl.pallas_call(
        _ab_body,
        grid=(grid,),
        in_specs=[
            pl.BlockSpec((blk, D), lambda i: (i, 0)),
            pl.BlockSpec((D, D), lambda i: (0, 0)),
            pl.BlockSpec((D, D), lambda i: (0, 0)),
        ],
        out_specs=[
            pl.BlockSpec((blk, D), lambda i: (i, 0)),
            pl.BlockSpec((blk, D), lambda i: (i, 0)),
        ],
        out_shape=[
            jax.ShapeDtypeStruct((N_NODES, D), jnp.float32),
            jax.ShapeDtypeStruct((N_NODES, D), jnp.float32),
        ],
    )(node_emb, w1a, w1b)


@functools.partial(
    pl.kernel,
    out_type=[
        jax.ShapeDtypeStruct((E, D), jnp.float32),
        jax.ShapeDtypeStruct((NW, N_PAD), jnp.float32),
    ],
    mesh=_MESH,
    compiler_params=pltpu.CompilerParams(needs_layout_passes=False),
    scratch_types=[
        pltpu.VMEM((EPW,), jnp.int32),
        pltpu.VMEM((EPW,), jnp.int32),
        pltpu.VMEM((CHUNK, D), jnp.float32),
        pltpu.VMEM((CHUNK, D), jnp.float32),
        pltpu.VMEM((CHUNK, D), jnp.float32),
        pltpu.VMEM((CHUNK, D), jnp.float32),
        pltpu.VMEM((N_PAD,), jnp.float32),
        pltpu.SemaphoreType.DMA,
        pltpu.SemaphoreType.DMA,
        pltpu.SemaphoreType.DMA,
        pltpu.SemaphoreType.DMA,
    ],
)
def _gather_kernel(a_hbm, b_hbm, src_hbm, dst_hbm, out_hbm, hist_hbm,
                   si, di, ba0, bb0, ba1, bb1, hist,
                   sa0, sb0, sa1, sb1):
    wid = lax.axis_index("s") * NC + lax.axis_index("c")
    base = wid * EPW

    pltpu.sync_copy(src_hbm.at[pl.ds(base, EPW)], si)
    pltpu.sync_copy(dst_hbm.at[pl.ds(base, EPW)], di)

    zeros16 = jnp.zeros((LANES,), jnp.float32)
    ones16 = jnp.ones((LANES,), jnp.float32)

    @pl.loop(0, N_PAD // LANES)
    def _zh(k):
        hist[pl.ds(k * LANES, LANES)] = zeros16

    def issue(g, ba, bb, sa, sb):
        lo = g * CHUNK
        pltpu.async_copy(a_hbm.at[si.at[pl.ds(lo, CHUNK)]], ba, sa)
        pltpu.async_copy(b_hbm.at[di.at[pl.ds(lo, CHUNK)]], bb, sb)

    def drain(ba, bb, sa, sb):
        pltpu.make_async_copy(a_hbm.at[pl.ds(0, CHUNK)], ba, sa).wait()
        pltpu.make_async_copy(b_hbm.at[pl.ds(0, CHUNK)], bb, sb).wait()

    def process(g, ba, bb):
        lo = g * CHUNK

        @pl.loop(0, CHUNK)
        def _row(r):
            for j in range(D // LANES):
                sl = pl.ds(j * LANES, LANES)
                ba[r, sl] = ba[r, sl] + bb[r, sl]

        for p in range(CHUNK // LANES):
            iv = si[pl.ds(lo + p * LANES, LANES)]
            plsc.addupdate_scatter(hist, [iv], ones16)

        pltpu.sync_copy(ba, out_hbm.at[pl.ds(base + lo, CHUNK)])

    issue(0, ba0, bb0, sa0, sb0)

    @pl.loop(0, (NCHUNK - 1) // 2)
    def _go(go):
        g0 = go * 2
        issue(g0 + 1, ba1, bb1, sa1, sb1)
        drain(ba0, bb0, sa0, sb0)
        process(g0, ba0, bb0)
        issue(g0 + 2, ba0, bb0, sa0, sb0)
        drain(ba1, bb1, sa1, sb1)
        process(g0 + 1, ba1, bb1)

    drain(ba0, bb0, sa0, sb0)
    process(NCHUNK - 1, ba0, bb0)

    pltpu.sync_copy(hist, hist_hbm.at[wid])


def _mlp_body(h0_ref, b1_ref, w2_ref, b2_ref, wg_ref, bg_ref, out_ref):
    h = jnp.maximum(h0_ref[...] + b1_ref[...], 0.0)
    inter = jnp.dot(h, w2_ref[...], preferred_element_type=jnp.float32) + b2_ref[...]
    gate = jax.nn.sigmoid(
        jnp.dot(inter, wg_ref[...], preferred_element_type=jnp.float32) + bg_ref[...]
    )
    out_ref[...] = inter * gate


def _edge_mlp(h0, b1, w2, b2, wg, bg):
    blk = 3200
    grid = E // blk
    vec = lambda i: (0, 0)
    return pl.pallas_call(
        _mlp_body,
        grid=(grid,),
        in_specs=[
            pl.BlockSpec((blk, D), lambda i: (i, 0)),
            pl.BlockSpec((1, D), vec),
            pl.BlockSpec((D, D), vec),
            pl.BlockSpec((1, D), vec),
            pl.BlockSpec((D, D), vec),
            pl.BlockSpec((1, D), vec),
        ],
        out_specs=pl.BlockSpec((blk, D), lambda i: (i, 0)),
        out_shape=jax.ShapeDtypeStruct((E, D), jnp.float32),
    )(h0, b1.reshape(1, D), w2, b2.reshape(1, D), wg, bg.reshape(1, D))


@functools.partial(
    pl.kernel,
    out_type=jax.ShapeDtypeStruct((NC, N_PAD, D), jnp.float32),
    mesh=_MESH,
    scratch_types=[
        pltpu.VMEM((CHUNK,), jnp.int32),
        pltpu.VMEM((CHUNK, D), jnp.float32),
        pltpu.VMEM((SUBR, D), jnp.float32),
        pltpu.VMEM_SHARED((N_PAD, D), jnp.float32),
    ],
)
def _scatter_kernel(gated_hbm, src_hbm, psum_hbm, idx_v, rows_v, stage_v, accum):
    cid = lax.axis_index("c")
    sid = lax.axis_index("s")
    wid = sid * NC + cid
    rbase = sid * RPT

    zeros16 = jnp.zeros((LANES,), jnp.float32)

    @pl.loop(0, SUBR)
    def _zero(r):
        for j in range(D // LANES):
            stage_v[r, pl.ds(j * LANES, LANES)] = zeros16

    @pl.loop(0, NSUB)
    def _zinit(k):
        pltpu.sync_copy(stage_v, accum.at[pl.ds(rbase + k * SUBR, SUBR)])

    plsc.subcore_barrier()

    base = wid * EPW

    @pl.loop(0, NCHUNK)
    def _chunk(g):
        off = base + g * CHUNK
        pltpu.sync_copy(src_hbm.at[pl.ds(off, CHUNK)], idx_v)
        pltpu.sync_copy(gated_hbm.at[pl.ds(off, CHUNK)], rows_v)
        pltpu.sync_copy(rows_v, accum.at[idx_v], add=True)

    plsc.subcore_barrier()

    @pl.loop(0, NSUB)
    def _wb(k):
        r0 = rbase + k * SUBR
        pltpu.sync_copy(accum.at[pl.ds(r0, SUBR)], stage_v)
        pltpu.sync_copy(stage_v, psum_hbm.at[cid, pl.ds(r0, SUBR)])


def _fin_body(p0_ref, p1_ref, h_ref, out_ref):
    ones_w = jnp.ones((NW, 1), jnp.float32)
    cnt = jax.lax.dot_general(
        h_ref[...], ones_w, (((0,), (0,)), ((), ())),
        preferred_element_type=jnp.float32,
    )
    out_ref[...] = (p0_ref[...] + p1_ref[...]) / jnp.maximum(cnt, 1.0)


def _finalize(psum, hists):
    blk = 1024
    grid = N_PAD // blk
    return pl.pallas_call(
        _fin_body,
        grid=(grid,),
        in_specs=[
            pl.BlockSpec((blk, D), lambda i: (i, 0)),
            pl.BlockSpec((blk, D), lambda i: (i, 0)),
            pl.BlockSpec((NW, blk), lambda i: (0, i)),
        ],
        out_specs=pl.BlockSpec((blk, D), lambda i: (i, 0)),
        out_shape=jax.ShapeDtypeStruct((N_PAD, D), jnp.float32),
    )(psum[0], psum[1], hists)


def kernel(node_emb, edge_index, W1, b1, W2, b2, Wg, bg):
    src = edge_index[0]
    dst = edge_index[1]
    a, b = _node_proj(node_emb, W1[:D], W1[D:])
    h0, hists = _gather_kernel(a, b, src, dst)
    gated = _edge_mlp(h0, b1, W2, b2, Wg, bg)
    psum = _scatter_kernel(gated, src)
    return _finalize(psum, hists)[:N_NODES]


# speedup vs baseline: 6.3991x; 1.2587x over previous
import functools

import jax
import jax.numpy as jnp
from jax import lax
from jax.experimental import pallas as pl
from jax.experimental.pallas import tpu as pltpu
from jax.experimental.pallas import tpu_sc as plsc

N_NODES = 10000
E = 320000
D = 128
LANES = 16

NC, NS = 2, 16
NW = NC * NS
EPW = E // NW
CHUNK = 80
NCHUNK = EPW // CHUNK
N_PAD = 10240
RPT = N_PAD // NS
SUBR = 64
NSUB = RPT // SUBR
CW = 128

_MESH = plsc.VectorSubcoreMesh(
    core_axis_name="c", subcore_axis_name="s", num_cores=NC, num_subcores=NS
)


def _ab_body(x_ref, w1a_ref, w1b_ref, a_ref, b_ref):
    x = x_ref[...]
    a_ref[...] = jnp.dot(x, w1a_ref[...], preferred_element_type=jnp.float32)
    b_ref[...] = jnp.dot(x, w1b_ref[...], preferred_element_type=jnp.float32)


def _node_proj(node_emb, w1a, w1b):
    blk = 1000
    grid = N_NODES // blk
    return pl.pallas_call(
        _ab_body,
        grid=(grid,),
        in_specs=[
            pl.BlockSpec((blk, D), lambda i: (i, 0)),
            pl.BlockSpec((D, D), lambda i: (0, 0)),
            pl.BlockSpec((D, D), lambda i: (0, 0)),
        ],
        out_specs=[
            pl.BlockSpec((blk, D), lambda i: (i, 0)),
            pl.BlockSpec((blk, D), lambda i: (i, 0)),
        ],
        out_shape=[
            jax.ShapeDtypeStruct((N_NODES, D), jnp.float32),
            jax.ShapeDtypeStruct((N_NODES, D), jnp.float32),
        ],
    )(node_emb, w1a, w1b)


@functools.partial(
    pl.kernel,
    out_type=[
        jax.ShapeDtypeStruct((E, D), jnp.float32),
        jax.ShapeDtypeStruct((NW, N_PAD), jnp.float32),
    ],
    mesh=_MESH,
    compiler_params=pltpu.CompilerParams(needs_layout_passes=False),
    scratch_types=[
        pltpu.VMEM((EPW,), jnp.int32),
        pltpu.VMEM((EPW,), jnp.int32),
        pltpu.VMEM((CHUNK, D), jnp.float32),
        pltpu.VMEM((CHUNK, D), jnp.float32),
        pltpu.VMEM((CHUNK, D), jnp.float32),
        pltpu.VMEM((CHUNK, D), jnp.float32),
        pltpu.VMEM((N_PAD,), jnp.float32),
        pltpu.SemaphoreType.DMA,
        pltpu.SemaphoreType.DMA,
        pltpu.SemaphoreType.DMA,
        pltpu.SemaphoreType.DMA,
    ],
)
def _gather_kernel(a_hbm, b_hbm, src_hbm, dst_hbm, out_hbm, hist_hbm,
                   si, di, ba0, bb0, ba1, bb1, hist,
                   sa0, sb0, sa1, sb1):
    wid = lax.axis_index("s") * NC + lax.axis_index("c")
    base = wid * EPW

    pltpu.sync_copy(src_hbm.at[pl.ds(base, EPW)], si)
    pltpu.sync_copy(dst_hbm.at[pl.ds(base, EPW)], di)

    zeros16 = jnp.zeros((LANES,), jnp.float32)
    ones16 = jnp.ones((LANES,), jnp.float32)

    @pl.loop(0, N_PAD // LANES)
    def _zh(k):
        hist[pl.ds(k * LANES, LANES)] = zeros16

    def issue(g, ba, bb, sa, sb):
        lo = g * CHUNK
        pltpu.async_copy(a_hbm.at[si.at[pl.ds(lo, CHUNK)]], ba, sa)
        pltpu.async_copy(b_hbm.at[di.at[pl.ds(lo, CHUNK)]], bb, sb)

    def drain(ba, bb, sa, sb):
        pltpu.make_async_copy(a_hbm.at[pl.ds(0, CHUNK)], ba, sa).wait()
        pltpu.make_async_copy(b_hbm.at[pl.ds(0, CHUNK)], bb, sb).wait()

    def process(g, ba, bb):
        lo = g * CHUNK

        @pl.loop(0, CHUNK)
        def _row(r):
            for j in range(D // LANES):
                sl = pl.ds(j * LANES, LANES)
                ba[r, sl] = ba[r, sl] + bb[r, sl]

        for p in range(CHUNK // LANES):
            iv = si[pl.ds(lo + p * LANES, LANES)]
            plsc.addupdate_scatter(hist, [iv], ones16)

        pltpu.sync_copy(ba, out_hbm.at[pl.ds(base + lo, CHUNK)])

    issue(0, ba0, bb0, sa0, sb0)

    @pl.loop(0, (NCHUNK - 1) // 2)
    def _go(go):
        g0 = go * 2
        issue(g0 + 1, ba1, bb1, sa1, sb1)
        drain(ba0, bb0, sa0, sb0)
        process(g0, ba0, bb0)
        issue(g0 + 2, ba0, bb0, sa0, sb0)
        drain(ba1, bb1, sa1, sb1)
        process(g0 + 1, ba1, bb1)

    drain(ba0, bb0, sa0, sb0)
    process(NCHUNK - 1, ba0, bb0)

    pltpu.sync_copy(hist, hist_hbm.at[wid])


def _mlp_body(h0_ref, b1_ref, w2_ref, b2_ref, wg_ref, bg_ref, out_ref):
    h = jnp.maximum(h0_ref[...] + b1_ref[...], 0.0)
    inter = jnp.dot(h, w2_ref[...], preferred_element_type=jnp.float32) + b2_ref[...]
    gate = jax.nn.sigmoid(
        jnp.dot(inter, wg_ref[...], preferred_element_type=jnp.float32) + bg_ref[...]
    )
    out_ref[...] = inter * gate


def _edge_mlp(h0, b1, w2, b2, wg, bg):
    blk = 3200
    grid = E // blk
    vec = lambda i: (0, 0)
    return pl.pallas_call(
        _mlp_body,
        grid=(grid,),
        in_specs=[
            pl.BlockSpec((blk, D), lambda i: (i, 0)),
            pl.BlockSpec((1, D), vec),
            pl.BlockSpec((D, D), vec),
            pl.BlockSpec((1, D), vec),
            pl.BlockSpec((D, D), vec),
            pl.BlockSpec((1, D), vec),
        ],
        out_specs=pl.BlockSpec((blk, D), lambda i: (i, 0)),
        out_shape=jax.ShapeDtypeStruct((E, D), jnp.float32),
    )(h0, b1.reshape(1, D), w2, b2.reshape(1, D), wg, bg.reshape(1, D))


@functools.partial(
    pl.kernel,
    out_type=jax.ShapeDtypeStruct((NC, N_PAD, D), jnp.float32),
    mesh=_MESH,
    scratch_types=[
        pltpu.VMEM((NCHUNK, CHUNK), jnp.int32),
        pltpu.VMEM((CHUNK, D), jnp.float32),
        pltpu.VMEM((CHUNK, D), jnp.float32),
        pltpu.VMEM((SUBR, D), jnp.float32),
        pltpu.VMEM_SHARED((N_PAD, D), jnp.float32),
        pltpu.SemaphoreType.DMA,
        pltpu.SemaphoreType.DMA,
    ],
)
def _scatter_kernel(gated_hbm, src2d_hbm, psum_hbm,
                    idx_v, rows0, rows1, stage_v, accum, s0, s1):
    cid = lax.axis_index("c")
    sid = lax.axis_index("s")
    wid = sid * NC + cid
    rbase = sid * RPT
    base = wid * EPW

    pltpu.sync_copy(src2d_hbm.at[wid], idx_v)

    zeros16 = jnp.zeros((LANES,), jnp.float32)

    @pl.loop(0, SUBR)
    def _zero(r):
        for j in range(D // LANES):
            stage_v[r, pl.ds(j * LANES, LANES)] = zeros16

    @pl.loop(0, NSUB)
    def _zinit(k):
        pltpu.sync_copy(stage_v, accum.at[pl.ds(rbase + k * SUBR, SUBR)])

    plsc.subcore_barrier()

    def issue(g, rows, sem):
        pltpu.async_copy(gated_hbm.at[pl.ds(base + g * CHUNK, CHUNK)], rows, sem)

    def drain(rows, sem):
        pltpu.make_async_copy(gated_hbm.at[pl.ds(0, CHUNK)], rows, sem).wait()

    def scat(g, rows):
        pltpu.sync_copy(rows, accum.at[idx_v.at[g]], add=True)

    issue(0, rows0, s0)

    @pl.loop(0, (NCHUNK - 1) // 2)
    def _go(go):
        g0 = go * 2
        issue(g0 + 1, rows1, s1)
        drain(rows0, s0)
        scat(g0, rows0)
        issue(g0 + 2, rows0, s0)
        drain(rows1, s1)
        scat(g0 + 1, rows1)

    drain(rows0, s0)
    scat(NCHUNK - 1, rows0)

    plsc.subcore_barrier()

    @pl.loop(0, NSUB)
    def _wb(k):
        r0 = rbase + k * SUBR
        pltpu.sync_copy(accum.at[pl.ds(r0, SUBR)], stage_v)
        pltpu.sync_copy(stage_v, psum_hbm.at[cid, pl.ds(r0, SUBR)])


def _fin_body(p0_ref, p1_ref, h_ref, out_ref):
    ones_w = jnp.ones((NW, 1), jnp.float32)
    cnt = jax.lax.dot_general(
        h_ref[...], ones_w, (((0,), (0,)), ((), ())),
        preferred_element_type=jnp.float32,
    )
    out_ref[...] = (p0_ref[...] + p1_ref[...]) / jnp.maximum(cnt, 1.0)


def _finalize(psum, hists):
    blk = 1024
    grid = N_PAD // blk
    return pl.pallas_call(
        _fin_body,
        grid=(grid,),
        in_specs=[
            pl.BlockSpec((blk, D), lambda i: (i, 0)),
            pl.BlockSpec((blk, D), lambda i: (i, 0)),
            pl.BlockSpec((NW, blk), lambda i: (0, i)),
        ],
        out_specs=pl.BlockSpec((blk, D), lambda i: (i, 0)),
        out_shape=jax.ShapeDtypeStruct((N_PAD, D), jnp.float32),
    )(psum[0], psum[1], hists)


def kernel(node_emb, edge_index, W1, b1, W2, b2, Wg, bg):
    src = edge_index[0]
    dst = edge_index[1]
    a, b = _node_proj(node_emb, W1[:D], W1[D:])
    h0, hists = _gather_kernel(a, b, src, dst)
    gated = _edge_mlp(h0, b1, W2, b2, Wg, bg)
    psum = _scatter_kernel(gated, src.reshape(NW, NCHUNK, CHUNK))
    return _finalize(psum, hists)[:N_NODES]


# speedup vs baseline: 7.0195x; 1.0969x over previous
import functools

import jax
import jax.numpy as jnp
from jax import lax
from jax.experimental import pallas as pl
from jax.experimental.pallas import tpu as pltpu
from jax.experimental.pallas import tpu_sc as plsc

N_NODES = 10000
E = 320000
D = 128
LANES = 16

NC, NS = 2, 16
NW = NC * NS
EPW = E // NW
CHUNK = 80
NCHUNK = EPW // CHUNK
N_PAD = 10240
RPT = N_PAD // NS
SUBR = 64
NSUB = RPT // SUBR
NCH1 = 64
NCH2 = NCHUNK - NCH1

_MESH = plsc.VectorSubcoreMesh(
    core_axis_name="c", subcore_axis_name="s", num_cores=NC, num_subcores=NS
)


def _ab_body(x_ref, w1a_ref, w1b_ref, a_ref, b_ref):
    x = x_ref[...]
    a_ref[...] = jnp.dot(x, w1a_ref[...], preferred_element_type=jnp.float32)
    b_ref[...] = jnp.dot(x, w1b_ref[...], preferred_element_type=jnp.float32)


def _node_proj(node_emb, w1a, w1b):
    blk = 1000
    grid = N_NODES // blk
    return pl.pallas_call(
        _ab_body,
        grid=(grid,),
        in_specs=[
            pl.BlockSpec((blk, D), lambda i: (i, 0)),
            pl.BlockSpec((D, D), lambda i: (0, 0)),
            pl.BlockSpec((D, D), lambda i: (0, 0)),
        ],
        out_specs=[
            pl.BlockSpec((blk, D), lambda i: (i, 0)),
            pl.BlockSpec((blk, D), lambda i: (i, 0)),
        ],
        out_shape=[
            jax.ShapeDtypeStruct((N_NODES, D), jnp.float32),
            jax.ShapeDtypeStruct((N_NODES, D), jnp.float32),
        ],
    )(node_emb, w1a, w1b)


def _make_gather(ch_lo, n_ch):
    epw_h = n_ch * CHUNK
    e_h = NW * epw_h

    @functools.partial(
        pl.kernel,
        out_type=[
            jax.ShapeDtypeStruct((e_h, D), jnp.float32),
            jax.ShapeDtypeStruct((NW, N_PAD), jnp.float32),
        ],
        mesh=_MESH,
        compiler_params=pltpu.CompilerParams(needs_layout_passes=False),
        scratch_types=[
            pltpu.VMEM((epw_h,), jnp.int32),
            pltpu.VMEM((epw_h,), jnp.int32),
            pltpu.VMEM((CHUNK, D), jnp.float32),
            pltpu.VMEM((CHUNK, D), jnp.float32),
            pltpu.VMEM((CHUNK, D), jnp.float32),
            pltpu.VMEM((CHUNK, D), jnp.float32),
            pltpu.VMEM((N_PAD,), jnp.float32),
            pltpu.SemaphoreType.DMA,
            pltpu.SemaphoreType.DMA,
            pltpu.SemaphoreType.DMA,
            pltpu.SemaphoreType.DMA,
        ],
    )
    def gather(a_hbm, b_hbm, src_hbm, dst_hbm, dep_hbm, out_hbm, hist_hbm,
               si, di, ba0, bb0, ba1, bb1, hist,
               sa0, sb0, sa1, sb1):
        del dep_hbm
        wid = lax.axis_index("s") * NC + lax.axis_index("c")
        base_in = wid * EPW + ch_lo * CHUNK
        base_out = wid * epw_h

        pltpu.sync_copy(src_hbm.at[pl.ds(base_in, epw_h)], si)
        pltpu.sync_copy(dst_hbm.at[pl.ds(base_in, epw_h)], di)

        zeros16 = jnp.zeros((LANES,), jnp.float32)
        ones16 = jnp.ones((LANES,), jnp.float32)

        @pl.loop(0, N_PAD // LANES)
        def _zh(k):
            hist[pl.ds(k * LANES, LANES)] = zeros16

        def issue(g, ba, bb, sa, sb):
            lo = g * CHUNK
            pltpu.async_copy(a_hbm.at[si.at[pl.ds(lo, CHUNK)]], ba, sa)
            pltpu.async_copy(b_hbm.at[di.at[pl.ds(lo, CHUNK)]], bb, sb)

        def drain(ba, bb, sa, sb):
            pltpu.make_async_copy(a_hbm.at[pl.ds(0, CHUNK)], ba, sa).wait()
            pltpu.make_async_copy(b_hbm.at[pl.ds(0, CHUNK)], bb, sb).wait()

        def process(g, ba, bb):
            lo = g * CHUNK

            @pl.loop(0, CHUNK)
            def _row(r):
                for j in range(D // LANES):
                    sl = pl.ds(j * LANES, LANES)
                    ba[r, sl] = ba[r, sl] + bb[r, sl]

            for p in range(CHUNK // LANES):
                iv = si[pl.ds(lo + p * LANES, LANES)]
                plsc.addupdate_scatter(hist, [iv], ones16)

            pltpu.sync_copy(ba, out_hbm.at[pl.ds(base_out + lo, CHUNK)])

        issue(0, ba0, bb0, sa0, sb0)

        @pl.loop(0, (n_ch - 1) // 2)
        def _go(go):
            g0 = go * 2
            issue(g0 + 1, ba1, bb1, sa1, sb1)
            drain(ba0, bb0, sa0, sb0)
            process(g0, ba0, bb0)
            issue(g0 + 2, ba0, bb0, sa0, sb0)
            drain(ba1, bb1, sa1, sb1)
            process(g0 + 1, ba1, bb1)

        if n_ch % 2 == 0:
            issue(n_ch - 1, ba1, bb1, sa1, sb1)
            drain(ba0, bb0, sa0, sb0)
            process(n_ch - 2, ba0, bb0)
            drain(ba1, bb1, sa1, sb1)
            process(n_ch - 1, ba1, bb1)
        else:
            drain(ba0, bb0, sa0, sb0)
            process(n_ch - 1, ba0, bb0)

        pltpu.sync_copy(hist, hist_hbm.at[wid])

    return gather


_gather1 = _make_gather(0, NCH1)
_gather2 = _make_gather(NCH1, NCH2)


def _mlp_body(h0_ref, b1_ref, w2_ref, b2_ref, wg_ref, bg_ref, out_ref):
    h = jnp.maximum(h0_ref[...] + b1_ref[...], 0.0)
    inter = jnp.dot(h, w2_ref[...], preferred_element_type=jnp.float32) + b2_ref[...]
    gate = jax.nn.sigmoid(
        jnp.dot(inter, wg_ref[...], preferred_element_type=jnp.float32) + bg_ref[...]
    )
    out_ref[...] = inter * gate


def _edge_mlp(h0, b1, w2, b2, wg, bg):
    e_h = h0.shape[0]
    blk = 2560
    grid = e_h // blk
    vec = lambda i: (0, 0)
    return pl.pallas_call(
        _mlp_body,
        grid=(grid,),
        in_specs=[
            pl.BlockSpec((blk, D), lambda i: (i, 0)),
            pl.BlockSpec((1, D), vec),
            pl.BlockSpec((D, D), vec),
            pl.BlockSpec((1, D), vec),
            pl.BlockSpec((D, D), vec),
            pl.BlockSpec((1, D), vec),
        ],
        out_specs=pl.BlockSpec((blk, D), lambda i: (i, 0)),
        out_shape=jax.ShapeDtypeStruct((e_h, D), jnp.float32),
    )(h0, b1.reshape(1, D), w2, b2.reshape(1, D), wg, bg.reshape(1, D))


def _make_scatter(ch_lo, n_ch):
    epw_h = n_ch * CHUNK

    @functools.partial(
        pl.kernel,
        out_type=jax.ShapeDtypeStruct((NC, N_PAD, D), jnp.float32),
        mesh=_MESH,
        scratch_types=[
            pltpu.VMEM((n_ch, CHUNK), jnp.int32),
            pltpu.VMEM((CHUNK, D), jnp.float32),
            pltpu.VMEM((CHUNK, D), jnp.float32),
            pltpu.VMEM((SUBR, D), jnp.float32),
            pltpu.VMEM_SHARED((N_PAD, D), jnp.float32),
            pltpu.SemaphoreType.DMA,
            pltpu.SemaphoreType.DMA,
        ],
    )
    def scatter(gated_hbm, src2d_hbm, dep_hbm, psum_hbm,
                idx_v, rows0, rows1, stage_v, accum, s0, s1):
        del dep_hbm
        cid = lax.axis_index("c")
        sid = lax.axis_index("s")
        wid = sid * NC + cid
        rbase = sid * RPT
        base = wid * epw_h

        pltpu.sync_copy(src2d_hbm.at[wid, pl.ds(ch_lo, n_ch)], idx_v)

        zeros16 = jnp.zeros((LANES,), jnp.float32)

        @pl.loop(0, SUBR)
        def _zero(r):
            for j in range(D // LANES):
                stage_v[r, pl.ds(j * LANES, LANES)] = zeros16

        @pl.loop(0, NSUB)
        def _zinit(k):
            pltpu.sync_copy(stage_v, accum.at[pl.ds(rbase + k * SUBR, SUBR)])

        plsc.subcore_barrier()

        def issue(g, rows, sem):
            pltpu.async_copy(
                gated_hbm.at[pl.ds(base + g * CHUNK, CHUNK)], rows, sem
            )

        def drain(rows, sem):
            pltpu.make_async_copy(
                gated_hbm.at[pl.ds(0, CHUNK)], rows, sem
            ).wait()

        def scat(g, rows):
            pltpu.sync_copy(rows, accum.at[idx_v.at[g]], add=True)

        issue(0, rows0, s0)

        @pl.loop(0, (n_ch - 1) // 2)
        def _go(go):
            g0 = go * 2
            issue(g0 + 1, rows1, s1)
            drain(rows0, s0)
            scat(g0, rows0)
            issue(g0 + 2, rows0, s0)
            drain(rows1, s1)
            scat(g0 + 1, rows1)

        if n_ch % 2 == 0:
            issue(n_ch - 1, rows1, s1)
            drain(rows0, s0)
            scat(n_ch - 2, rows0)
            drain(rows1, s1)
            scat(n_ch - 1, rows1)
        else:
            drain(rows0, s0)
            scat(n_ch - 1, rows0)

        plsc.subcore_barrier()

        @pl.loop(0, NSUB)
        def _wb(k):
            r0 = rbase + k * SUBR
            pltpu.sync_copy(accum.at[pl.ds(r0, SUBR)], stage_v)
            pltpu.sync_copy(stage_v, psum_hbm.at[cid, pl.ds(r0, SUBR)])

    return scatter


_scatter1 = _make_scatter(0, NCH1)
_scatter2 = _make_scatter(NCH1, NCH2)


def _fin_body(p10_ref, p11_ref, p20_ref, p21_ref, h_ref, out_ref):
    ones_w = jnp.ones((2 * NW, 1), jnp.float32)
    cnt = jax.lax.dot_general(
        h_ref[...], ones_w, (((0,), (0,)), ((), ())),
        preferred_element_type=jnp.float32,
    )
    s = (p10_ref[...] + p11_ref[...]) + (p20_ref[...] + p21_ref[...])
    out_ref[...] = s / jnp.maximum(cnt, 1.0)


def _finalize(psum1, psum2, hists):
    blk = 1024
    grid = N_PAD // blk
    blk_spec = pl.BlockSpec((blk, D), lambda i: (i, 0))
    return pl.pallas_call(
        _fin_body,
        grid=(grid,),
        in_specs=[
            blk_spec,
            blk_spec,
            blk_spec,
            blk_spec,
            pl.BlockSpec((2 * NW, blk), lambda i: (0, i)),
        ],
        out_specs=blk_spec,
        out_shape=jax.ShapeDtypeStruct((N_PAD, D), jnp.float32),
    )(psum1[0], psum1[1], psum2[0], psum2[1], hists)


def kernel(node_emb, edge_index, W1, b1, W2, b2, Wg, bg):
    src = edge_index[0]
    dst = edge_index[1]
    src2d = src.reshape(NW, NCHUNK, CHUNK)
    a, b = _node_proj(node_emb, W1[:D], W1[D:])
    h0_1, hist1 = _gather1(a, b, src, dst, a)
    h0_2, hist2 = _gather2(a, b, src, dst, hist1)
    gated1 = _edge_mlp(h0_1, b1, W2, b2, Wg, bg)
    gated2 = _edge_mlp(h0_2, b1, W2, b2, Wg, bg)
    psum1 = _scatter1(gated1, src2d, hist2)
    psum2 = _scatter2(gated2, src2d, psum1)
    hists = jnp.concatenate([hist1, hist2], axis=0)
    return _finalize(psum1, psum2, hists)[:N_NODES]
